# baseline (speedup 1.0000x reference)
import jax
import jax.numpy as jnp
from jax.experimental import pallas as pl
from jax.experimental.pallas import tpu as pltpu

_N, _D, _H, _E, _K = 2048, 768, 768, 8, 2


_BT = _N // 2


def _moe_body(x_ref, wg_ref, bg_ref, w1_ref, b1_ref, w2_ref, b2_ref,
              out_ref, gate):
    e = pl.program_id(0)
    t = pl.program_id(1)
    rows = pl.ds(t * _BT, _BT)

    @pl.when((e == 0) & (t == 0))
    def _gate():
        x = x_ref[...]
        scores = jnp.dot(x, wg_ref[...],
                         preferred_element_type=jnp.float32) + bg_ref[...]
        eidx = jax.lax.broadcasted_iota(jnp.int32, scores.shape, 1)
        m1 = jnp.max(scores, axis=1, keepdims=True)
        i1 = jnp.min(jnp.where(scores == m1, eidx, _E), axis=1, keepdims=True)
        oh1 = eidx == i1
        neg = jnp.where(oh1, -jnp.inf, scores)
        m2 = jnp.max(neg, axis=1, keepdims=True)
        i2 = jnp.min(jnp.where(neg == m2, eidx, _E), axis=1, keepdims=True)
        oh2 = eidx == i2
        ex = jnp.exp(scores - m1)
        p = ex / jnp.sum(ex, axis=1, keepdims=True)
        wm = p * (oh1 | oh2).astype(jnp.float32)
        gate[...] = wm / (jnp.sum(wm, axis=1, keepdims=True) + 1e-8)

    x = x_ref[rows, :]
    h = jnp.maximum(
        jnp.dot(x, w1_ref[0], preferred_element_type=jnp.float32) + b1_ref[0],
        0.0)
    o = jnp.dot(h, w2_ref[0], preferred_element_type=jnp.float32) + b2_ref[0]
    ge = gate[rows, :]
    sel = (jax.lax.broadcasted_iota(jnp.int32, ge.shape, 1) == e)
    wcol = jnp.sum(jnp.where(sel, ge, 0.0), axis=1, keepdims=True)
    contrib = wcol * o

    @pl.when(e == 0)
    def _init():
        out_ref[rows, :] = contrib

    @pl.when(e > 0)
    def _acc():
        out_ref[rows, :] += contrib


def kernel(x, Wg, bg, W1, b1, W2, b2):
    return pl.pallas_call(
        _moe_body,
        grid=(_E, 2),
        in_specs=[
            pl.BlockSpec((_N, _D), lambda e, t: (0, 0)),
            pl.BlockSpec((_D, _E), lambda e, t: (0, 0)),
            pl.BlockSpec((1, _E), lambda e, t: (0, 0)),
            pl.BlockSpec((1, _D, _H), lambda e, t: (e, 0, 0)),
            pl.BlockSpec((1, 1, _H), lambda e, t: (e, 0, 0)),
            pl.BlockSpec((1, _H, _D), lambda e, t: (e, 0, 0)),
            pl.BlockSpec((1, 1, _D), lambda e, t: (e, 0, 0)),
        ],
        out_specs=pl.BlockSpec((_N, _D), lambda e, t: (0, 0)),
        out_shape=jax.ShapeDtypeStruct((_N, _D), jnp.float32),
        scratch_shapes=[
            pltpu.VMEM((_N, _E), jnp.float32),
        ],
        compiler_params=pltpu.CompilerParams(
            dimension_semantics=("arbitrary", "arbitrary")),
    )(x, Wg, bg.reshape(1, _E), W1, b1.reshape(_E, 1, _H),
      W2, b2.reshape(_E, 1, _D))


# speedup vs baseline: 1.0052x; 1.0052x over previous
import jax
import jax.numpy as jnp
from jax.experimental import pallas as pl
from jax.experimental.pallas import tpu as pltpu

_N, _D, _H, _E, _K = 2048, 768, 768, 8, 2


def _moe_body(x_ref, wg_ref, bg_ref, w1_ref, b1_ref, w2_ref, b2_ref,
              out_ref, gate, xb):
    e = pl.program_id(0)

    @pl.when(e == 0)
    def _gate():
        x = x_ref[...]
        scores = jnp.dot(x, wg_ref[...],
                         preferred_element_type=jnp.float32) + bg_ref[...]
        eidx = jax.lax.broadcasted_iota(jnp.int32, scores.shape, 1)
        m1 = jnp.max(scores, axis=1, keepdims=True)
        i1 = jnp.min(jnp.where(scores == m1, eidx, _E), axis=1, keepdims=True)
        oh1 = eidx == i1
        neg = jnp.where(oh1, -jnp.inf, scores)
        m2 = jnp.max(neg, axis=1, keepdims=True)
        i2 = jnp.min(jnp.where(neg == m2, eidx, _E), axis=1, keepdims=True)
        oh2 = eidx == i2
        ex = jnp.exp(scores - m1)
        p = ex / jnp.sum(ex, axis=1, keepdims=True)
        wm = p * (oh1 | oh2).astype(jnp.float32)
        gate[...] = wm / (jnp.sum(wm, axis=1, keepdims=True) + 1e-8)

    @pl.when(e == 0)
    def _pack():
        xb[...] = x_ref[...].astype(jnp.bfloat16)

    h = jnp.maximum(
        jnp.dot(xb[...], w1_ref[0],
                preferred_element_type=jnp.float32) + b1_ref[0],
        0.0)
    o = jnp.dot(h, w2_ref[0], preferred_element_type=jnp.float32) + b2_ref[0]
    ge = gate[...]
    sel = (jax.lax.broadcasted_iota(jnp.int32, ge.shape, 1) == e)
    wcol = jnp.sum(jnp.where(sel, ge, 0.0), axis=1, keepdims=True)
    contrib = wcol * o

    @pl.when(e == 0)
    def _init():
        out_ref[...] = contrib

    @pl.when(e > 0)
    def _acc():
        out_ref[...] += contrib


def kernel(x, Wg, bg, W1, b1, W2, b2):
    return pl.pallas_call(
        _moe_body,
        grid=(_E,),
        in_specs=[
            pl.BlockSpec((_N, _D), lambda e: (0, 0)),
            pl.BlockSpec((_D, _E), lambda e: (0, 0)),
            pl.BlockSpec((1, _E), lambda e: (0, 0)),
            pl.BlockSpec((1, _D, _H), lambda e: (e, 0, 0)),
            pl.BlockSpec((1, 1, _H), lambda e: (e, 0, 0)),
            pl.BlockSpec((1, _H, _D), lambda e: (e, 0, 0)),
            pl.BlockSpec((1, 1, _D), lambda e: (e, 0, 0)),
        ],
        out_specs=pl.BlockSpec((_N, _D), lambda e: (0, 0)),
        out_shape=jax.ShapeDtypeStruct((_N, _D), jnp.float32),
        scratch_shapes=[
            pltpu.VMEM((_N, _E), jnp.float32),
            pltpu.VMEM((_N, _D), jnp.bfloat16),
        ],
        compiler_params=pltpu.CompilerParams(
            dimension_semantics=("arbitrary",)),
    )(x, Wg, bg.reshape(1, _E), W1, b1.reshape(_E, 1, _H),
      W2, b2.reshape(_E, 1, _D))


# speedup vs baseline: 1.0094x; 1.0041x over previous
import jax
import jax.numpy as jnp
from jax.experimental import pallas as pl
from jax.experimental.pallas import tpu as pltpu

_N, _D, _H, _E, _K = 2048, 768, 768, 8, 2


def _moe_body(x_ref, wg_ref, bg_ref, w1_ref, b1_ref, w2_ref, b2_ref,
              out_ref, gate, xb):
    e = pl.program_id(0)

    @pl.when(e == 0)
    def _gate():
        x = x_ref[...]
        scores = jnp.dot(x, wg_ref[...],
                         preferred_element_type=jnp.float32) + bg_ref[...]
        eidx = jax.lax.broadcasted_iota(jnp.int32, scores.shape, 1)
        m1 = jnp.max(scores, axis=1, keepdims=True)
        i1 = jnp.min(jnp.where(scores == m1, eidx, _E), axis=1, keepdims=True)
        oh1 = eidx == i1
        neg = jnp.where(oh1, -jnp.inf, scores)
        m2 = jnp.max(neg, axis=1, keepdims=True)
        i2 = jnp.min(jnp.where(neg == m2, eidx, _E), axis=1, keepdims=True)
        oh2 = eidx == i2
        ex = jnp.exp(scores - m1)
        p = ex / jnp.sum(ex, axis=1, keepdims=True)
        wm = p * (oh1 | oh2).astype(jnp.float32)
        gate[...] = wm / (jnp.sum(wm, axis=1, keepdims=True) + 1e-8)

    @pl.when(e == 0)
    def _pack():
        xb[...] = x_ref[...].astype(jnp.bfloat16)

    h = jnp.maximum(
        jnp.dot(xb[...], w1_ref[0].astype(jnp.bfloat16),
                preferred_element_type=jnp.float32) + b1_ref[0],
        0.0)
    o = jnp.dot(h.astype(jnp.bfloat16), w2_ref[0].astype(jnp.bfloat16),
                preferred_element_type=jnp.float32) + b2_ref[0]
    ge = gate[...]
    sel = (jax.lax.broadcasted_iota(jnp.int32, ge.shape, 1) == e)
    wcol = jnp.sum(jnp.where(sel, ge, 0.0), axis=1, keepdims=True)
    contrib = wcol * o

    @pl.when(e == 0)
    def _init():
        out_ref[...] = contrib

    @pl.when(e > 0)
    def _acc():
        out_ref[...] += contrib


def kernel(x, Wg, bg, W1, b1, W2, b2):
    return pl.pallas_call(
        _moe_body,
        grid=(_E,),
        in_specs=[
            pl.BlockSpec((_N, _D), lambda e: (0, 0)),
            pl.BlockSpec((_D, _E), lambda e: (0, 0)),
            pl.BlockSpec((1, _E), lambda e: (0, 0)),
            pl.BlockSpec((1, _D, _H), lambda e: (e, 0, 0)),
            pl.BlockSpec((1, 1, _H), lambda e: (e, 0, 0)),
            pl.BlockSpec((1, _H, _D), lambda e: (e, 0, 0)),
            pl.BlockSpec((1, 1, _D), lambda e: (e, 0, 0)),
        ],
        out_specs=pl.BlockSpec((_N, _D), lambda e: (0, 0)),
        out_shape=jax.ShapeDtypeStruct((_N, _D), jnp.float32),
        scratch_shapes=[
            pltpu.VMEM((_N, _E), jnp.float32),
            pltpu.VMEM((_N, _D), jnp.bfloat16),
        ],
        compiler_params=pltpu.CompilerParams(
            dimension_semantics=("arbitrary",)),
    )(x, Wg, bg.reshape(1, _E), W1, b1.reshape(_E, 1, _H),
      W2, b2.reshape(_E, 1, _D))


# speedup vs baseline: 1.0194x; 1.0099x over previous
import jax
import jax.numpy as jnp
from jax.experimental import pallas as pl
from jax.experimental.pallas import tpu as pltpu

_N, _D, _H, _E, _K = 2048, 768, 768, 8, 2


def _moe_body(x_ref, wg_ref, bg_ref, w1_ref, b1_ref, w2_ref, b2_ref,
              out_ref, gate):
    e = pl.program_id(0)

    @pl.when(e == 0)
    def _gate():
        x = x_ref[...]
        scores = jnp.dot(x, wg_ref[...],
                         preferred_element_type=jnp.float32) + bg_ref[...]
        eidx = jax.lax.broadcasted_iota(jnp.int32, scores.shape, 1)
        m1 = jnp.max(scores, axis=1, keepdims=True)
        i1 = jnp.min(jnp.where(scores == m1, eidx, _E), axis=1, keepdims=True)
        oh1 = eidx == i1
        neg = jnp.where(oh1, -jnp.inf, scores)
        m2 = jnp.max(neg, axis=1, keepdims=True)
        i2 = jnp.min(jnp.where(neg == m2, eidx, _E), axis=1, keepdims=True)
        oh2 = eidx == i2
        ex = jnp.exp(scores - m1)
        p = ex / jnp.sum(ex, axis=1, keepdims=True)
        wm = p * (oh1 | oh2).astype(jnp.float32)
        gate[...] = wm / (jnp.sum(wm, axis=1, keepdims=True) + 1e-8)

    h = jnp.maximum(
        jnp.dot(x_ref[...], w1_ref[0],
                preferred_element_type=jnp.float32) + b1_ref[0],
        0.0)
    o = jnp.dot(h, w2_ref[0], preferred_element_type=jnp.float32) + b2_ref[0]
    ge = gate[...]
    sel = (jax.lax.broadcasted_iota(jnp.int32, ge.shape, 1) == e)
    wcol = jnp.sum(jnp.where(sel, ge, 0.0), axis=1, keepdims=True)
    contrib = wcol * o

    @pl.when(e == 0)
    def _init():
        out_ref[...] = contrib

    @pl.when(e > 0)
    def _acc():
        out_ref[...] += contrib


def kernel(x, Wg, bg, W1, b1, W2, b2):
    return pl.pallas_call(
        _moe_body,
        grid=(_E,),
        in_specs=[
            pl.BlockSpec((_N, _D), lambda e: (0, 0)),
            pl.BlockSpec((_D, _E), lambda e: (0, 0)),
            pl.BlockSpec((1, _E), lambda e: (0, 0)),
            pl.BlockSpec((1, _D, _H), lambda e: (e, 0, 0)),
            pl.BlockSpec((1, 1, _H), lambda e: (e, 0, 0)),
            pl.BlockSpec((1, _H, _D), lambda e: (e, 0, 0)),
            pl.BlockSpec((1, 1, _D), lambda e: (e, 0, 0)),
        ],
        out_specs=pl.BlockSpec((_N, _D), lambda e: (0, 0)),
        out_shape=jax.ShapeDtypeStruct((_N, _D), jnp.float32),
        scratch_shapes=[
            pltpu.VMEM((_N, _E), jnp.float32),
        ],
        compiler_params=pltpu.CompilerParams(
            dimension_semantics=("arbitrary",)),
    )(x, Wg, bg.reshape(1, _E), W1, b1.reshape(_E, 1, _H),
      W2, b2.reshape(_E, 1, _D))
